# baseline (speedup 1.0000x reference)
import functools

import jax
import jax.numpy as jnp
from jax import lax
from jax.experimental import pallas as pl
from jax.experimental.pallas import tpu as pltpu
from jax.experimental.pallas import tpu_sc as plsc

_TI = 128
_DIAG_LEN = 4608
_HALF = _DIAG_LEN // 2
_THRESH = [16, 19, 21, 24, 27, 31, 35, 40, 46, 52, 59, 67, 77, 87, 99, 113]


def _make_diag_sc(t_len, num_heads, num_buckets):
    mesh = plsc.VectorSubcoreMesh(core_axis_name="c", subcore_axis_name="s")
    info = plsc.get_sparse_core_info()
    nc = info.num_cores

    @functools.partial(
        pl.kernel,
        mesh=mesh,
        out_type=jax.ShapeDtypeStruct((num_heads, _DIAG_LEN), jnp.float32),
        scratch_types=[
            pltpu.VMEM((num_buckets,), jnp.float32),
            pltpu.VMEM((16,), jnp.int32),
            pltpu.VMEM((16,), jnp.int32),
            pltpu.VMEM((_HALF,), jnp.float32),
        ],
    )
    def diag_kernel(tab_hbm, off_hbm, mb_hbm, out_hbm, tab_v, off_v, mb_v, buf_v):
        wid = lax.axis_index("s") * nc + lax.axis_index("c")
        h = wid // 2
        half = wid % 2
        pltpu.sync_copy(tab_hbm.at[h], tab_v)
        pltpu.sync_copy(off_hbm, off_v)
        pltpu.sync_copy(mb_hbm.at[half], mb_v)
        lo = tab_v[0:16]
        hi = tab_v[16:32]
        iota = lax.iota(jnp.int32, 16)
        e0 = (t_len - 1 + 128) + off_v[...] - mb_v[...] - iota
        for i in range(_HALF // 16):
            d = jnp.maximum(e0 - (16 * i), 0)
            cnt = jnp.full((16,), 15, jnp.int32)
            for th in _THRESH:
                cnt = cnt + jnp.where(d >= th, 1, 0)
            bucket = jnp.where(d < 16, d, cnt)
            v_lo = lo.at[jnp.minimum(bucket, 15)].get(mode="promise_in_bounds")
            v_hi = hi.at[jnp.maximum(bucket - 16, 0)].get(mode="promise_in_bounds")
            buf_v[pl.ds(16 * i, 16)] = jnp.where(bucket < 16, v_lo, v_hi)
        pltpu.sync_copy(buf_v, out_hbm.at[h, pl.ds(half * _HALF, _HALF)])

    return diag_kernel


def _expand_tile_kernel(diag_ref, out_ref, diag8_ref, *, t_len):
    ti = out_ref.shape[2]
    s_len = out_ref.shape[3]
    t = pl.program_id(1)
    w0 = (t_len - ti) - t * ti

    @pl.when(t == 0)
    def _():
        n = ((t_len + 2 * ti + s_len + 127) // 128) * 128 - ti
        big = diag_ref[0, 0:1, ti : ti + n + 128]
        for r in range(8):
            diag8_ref[r : r + 1, ti : ti + n] = jax.lax.slice(
                big, (0, 7 - r), (1, 7 - r + n)
            )

    win_len = ((120 + s_len + 127) // 128) * 128
    tile8 = diag8_ref[:, pl.ds(w0 + ti, win_len)]
    for g in range(ti // 8):
        x = 120 - 8 * g
        out_ref[0, 0, 8 * g : 8 * g + 8, :] = jax.lax.slice(
            tile8, (0, x), (8, x + s_len)
        )


def kernel(query, key, offset, rel_bias_table):
    t_len = query.shape[1]
    s_len = key.shape[1]
    num_buckets, num_heads = rel_bias_table.shape
    tab_t = rel_bias_table.T
    off = jnp.full((16,), offset, jnp.int32)
    mb = jnp.stack([jnp.zeros((16,), jnp.int32), jnp.full((16,), _HALF, jnp.int32)])

    diag = _make_diag_sc(t_len, num_heads, num_buckets)(tab_t, off, mb)
    diag = diag.reshape(num_heads, 1, _DIAG_LEN)

    body = functools.partial(_expand_tile_kernel, t_len=t_len)
    return pl.pallas_call(
        body,
        grid=(num_heads, t_len // _TI),
        in_specs=[
            pl.BlockSpec((1, 1, _DIAG_LEN), lambda h, t: (h, 0, 0)),
        ],
        out_specs=pl.BlockSpec((1, 1, _TI, s_len), lambda h, t: (0, h, t, 0)),
        out_shape=jax.ShapeDtypeStruct((1, num_heads, t_len, s_len), jnp.float32),
        scratch_shapes=[pltpu.VMEM((8, _DIAG_LEN), jnp.float32)],
    )(diag)


# speedup vs baseline: 1.0487x; 1.0487x over previous
import functools
import math

import jax
import jax.numpy as jnp
from jax import lax
from jax.experimental import pallas as pl
from jax.experimental.pallas import tpu as pltpu
from jax.experimental.pallas import tpu_sc as plsc

_TI = 128
_DIAG_LEN = 4608
_WHALF = _DIAG_LEN // 2
_THRESH = [16, 19, 21, 24, 27, 31, 35, 40, 46, 52, 59, 67, 77, 87, 99, 113]


def _bucket_py(d):
    if d < 16:
        return d
    return 15 + sum(1 for th in _THRESH if d >= th)


def _band_layout(t_len):
    es = t_len - 1 + 128
    kinds, band_rows = [], []
    for hf in (0, 1):
        row = []
        for i in range(_WHALF // 16):
            b = [
                _bucket_py(max(es - (hf * _WHALF + 16 * i + l), 0))
                for l in range(16)
            ]
            if all(x == 31 for x in b):
                row.append("v31")
            elif all(x == 0 for x in b):
                row.append("v0")
            else:
                row.append(len(band_rows))
                band_rows.append(
                    [min(x, 15) for x in b]
                    + [max(x - 16, 0) for x in b]
                    + [1 if x < 16 else 0 for x in b]
                )
        kinds.append(row)
    return kinds, band_rows


def _make_diag_sc(t_len, num_heads, num_buckets, kinds, n_band):
    mesh = plsc.VectorSubcoreMesh(core_axis_name="c", subcore_axis_name="s")

    @functools.partial(
        pl.kernel,
        mesh=mesh,
        out_type=jax.ShapeDtypeStruct((num_heads, _DIAG_LEN), jnp.float32),
        scratch_types=[
            pltpu.VMEM((num_buckets,), jnp.float32),
            pltpu.VMEM((n_band * 48,), jnp.int32),
            pltpu.VMEM((_WHALF,), jnp.float32),
        ],
    )
    def diag_kernel(tab_hbm, band_hbm, out_hbm, tab_v, band_v, buf_v):
        h = lax.axis_index("s")
        half = lax.axis_index("c")
        pltpu.sync_copy(tab_hbm.at[h], tab_v)
        pltpu.sync_copy(band_hbm, band_v)
        lo = tab_v[0:16]
        hi = tab_v[16:32]
        zeros = lax.iota(jnp.int32, 16) * 0
        v0 = lo.at[zeros].get(mode="promise_in_bounds")
        v31 = hi.at[zeros + 15].get(mode="promise_in_bounds")

        def fill_half(hf):
            for i in range(_WHALF // 16):
                kind = kinds[hf][i]
                if kind == "v31":
                    buf_v[pl.ds(16 * i, 16)] = v31
                elif kind == "v0":
                    buf_v[pl.ds(16 * i, 16)] = v0
                else:
                    p = kind * 48
                    idx_lo = band_v[pl.ds(p, 16)]
                    idx_hi = band_v[pl.ds(p + 16, 16)]
                    small = band_v[pl.ds(p + 32, 16)]
                    buf_v[pl.ds(16 * i, 16)] = jnp.where(
                        small == 1,
                        lo.at[idx_lo].get(mode="promise_in_bounds"),
                        hi.at[idx_hi].get(mode="promise_in_bounds"),
                    )

        @pl.when(half == 0)
        def _():
            fill_half(0)

        @pl.when(half == 1)
        def _():
            fill_half(1)

        pltpu.sync_copy(buf_v, out_hbm.at[h, pl.ds(half * _WHALF, _WHALF)])

    return diag_kernel


def _diag_vals(m0, n, t_len, off, tab_ref, h, num_buckets, max_distance):
    k = (m0 - 128) + jax.lax.broadcasted_iota(jnp.int32, (1, n), 1)
    d = jnp.maximum(t_len - 1 + off - k, 0)
    max_exact = num_buckets // 2
    is_small = d < max_exact
    d_f = d.astype(jnp.float32)
    large = max_exact + (
        jnp.log(d_f / max_exact)
        / math.log(max_distance / max_exact)
        * (num_buckets - max_exact)
    ).astype(jnp.int32)
    large = jnp.minimum(large, num_buckets - 1)
    bucket = jnp.where(is_small, d, large)
    val = jnp.zeros((1, n), jnp.float32)
    for b in range(num_buckets):
        val = jnp.where(bucket == b, tab_ref[b, h], val)
    return val


def _expand_tile_kernel(diag_ref, out_ref, diag8_ref, *, t_len):
    ti = out_ref.shape[2]
    s_len = out_ref.shape[3]
    t = pl.program_id(1)
    w0 = (t_len - ti) - t * ti

    @pl.when(t == 0)
    def _():
        n = ((t_len + 2 * ti + s_len + 127) // 128) * 128 - ti
        big = diag_ref[0, 0:1, ti : ti + n + 128]
        for r in range(8):
            diag8_ref[r : r + 1, ti : ti + n] = jax.lax.slice(
                big, (0, 7 - r), (1, 7 - r + n)
            )

    win_len = ((120 + s_len + 127) // 128) * 128
    tile8 = diag8_ref[:, pl.ds(w0 + ti, win_len)]
    for g in range(ti // 8):
        x = 120 - 8 * g
        out_ref[0, 0, 8 * g : 8 * g + 8, :] = jax.lax.slice(
            tile8, (0, x), (8, x + s_len)
        )


def _general_tile_kernel(off_ref, tab_ref, out_ref, diag_ref, diag8_ref, *,
                         t_len, num_buckets, max_distance):
    ti = out_ref.shape[2]
    s_len = out_ref.shape[3]
    h = pl.program_id(0)
    t = pl.program_id(1)
    w0 = (t_len - ti) - t * ti
    vals = functools.partial(
        _diag_vals,
        t_len=t_len,
        off=off_ref[0],
        tab_ref=tab_ref,
        h=h,
        num_buckets=num_buckets,
        max_distance=max_distance,
    )

    @pl.when(t == 0)
    def _():
        n = ((t_len + 2 * ti + s_len + 127) // 128) * 128 - ti
        diag_ref[0:1, ti : ti + n + 128] = vals(ti, n + 128)
        big = diag_ref[0:1, ti : ti + n + 128]
        for r in range(8):
            diag8_ref[r : r + 1, ti : ti + n] = jax.lax.slice(
                big, (0, 7 - r), (1, 7 - r + n)
            )

    win_len = ((120 + s_len + 127) // 128) * 128
    tile8 = diag8_ref[:, pl.ds(w0 + ti, win_len)]
    for g in range(ti // 8):
        x = 120 - 8 * g
        out_ref[0, 0, 8 * g : 8 * g + 8, :] = jax.lax.slice(
            tile8, (0, x), (8, x + s_len)
        )


def kernel(query, key, offset, rel_bias_table):
    t_len = query.shape[1]
    s_len = key.shape[1]
    num_buckets, num_heads = rel_bias_table.shape
    out_shape = jax.ShapeDtypeStruct((1, num_heads, t_len, s_len), jnp.float32)
    off = jnp.asarray(offset, jnp.int32)

    def sc_path():
        tab_t = rel_bias_table.T
        kinds, band_rows = _band_layout(t_len)
        band = jnp.asarray(band_rows, jnp.int32).reshape(-1)
        diag = _make_diag_sc(
            t_len, num_heads, num_buckets, kinds, len(band_rows)
        )(tab_t, band)
        diag = diag.reshape(num_heads, 1, _DIAG_LEN)
        body = functools.partial(_expand_tile_kernel, t_len=t_len)
        return pl.pallas_call(
            body,
            grid=(num_heads, t_len // _TI),
            in_specs=[pl.BlockSpec((1, 1, _DIAG_LEN), lambda h, t: (h, 0, 0))],
            out_specs=pl.BlockSpec((1, 1, _TI, s_len), lambda h, t: (0, h, t, 0)),
            out_shape=out_shape,
            scratch_shapes=[pltpu.VMEM((8, _DIAG_LEN), jnp.float32)],
        )(diag)

    def tc_path():
        body = functools.partial(
            _general_tile_kernel,
            t_len=t_len,
            num_buckets=num_buckets,
            max_distance=128,
        )
        return pl.pallas_call(
            body,
            grid=(num_heads, t_len // _TI),
            in_specs=[
                pl.BlockSpec(memory_space=pltpu.SMEM),
                pl.BlockSpec(memory_space=pltpu.SMEM),
            ],
            out_specs=pl.BlockSpec((1, 1, _TI, s_len), lambda h, t: (0, h, t, 0)),
            out_shape=out_shape,
            scratch_shapes=[
                pltpu.VMEM((1, _DIAG_LEN), jnp.float32),
                pltpu.VMEM((8, _DIAG_LEN), jnp.float32),
            ],
        )(off.reshape(1), rel_bias_table)

    return lax.cond(off == 0, sc_path, tc_path)
